# Initial kernel scaffold; baseline (speedup 1.0000x reference)
import jax
import jax.numpy as jnp
from jax.experimental import pallas as pl


def kernel(node_type, edge_index, distance, emb, conv_W1, cf_W1, cf_b1, cf_W2, cf_b2, conv_W2, conv_b2, conv_W3, conv_b3, d1_W, d1_b, d2_W, d2_b, cls_W, cls_b, atom_W, atom_b, prop_W, prop_b):
    raise NotImplementedError("write your pallas kernel here")


# speedup vs baseline: 1.3318x; 1.3318x over previous
import functools

import numpy as np

import jax
import jax.numpy as jnp
from jax import lax
from jax.experimental import pallas as pl
from jax.experimental.pallas import tpu as pltpu
from jax.experimental.pallas import tpu_sc as plsc

N = 50000
E = 800000
DIM = 64
TYPE_NUM = 100
N_CONV = 3
CUTOFF = 5.0
N_CENTERS = 5
GAP = CUTOFF / (N_CENTERS - 1)
_CENTERS_NP = np.linspace(0.0, CUTOFF, N_CENTERS).astype(np.float32)

NP = 51200
EP = 819200

_INTERPRET = False

BE = 3200
REAL_E_BLOCKS = E // BE


def _softplus_b05(x):
    return 2.0 * jnp.logaddexp(0.0, 0.5 * x)


def _h_body(d_ref, w1_ref, b1_ref, w2_ref, b2_ref, out_ref):
    e = pl.program_id(1)
    d = d_ref[...]
    rbf = jnp.concatenate(
        [jnp.exp((-1.0 / GAP) * (d - float(c)) ** 2) for c in _CENTERS_NP],
        axis=1)
    t = jnp.dot(rbf, w1_ref[0], preferred_element_type=jnp.float32) + b1_ref[0]
    t = _softplus_b05(t)
    h = jnp.dot(t, w2_ref[0], preferred_element_type=jnp.float32) + b2_ref[0]
    h = jnp.where(e < REAL_E_BLOCKS, h, 0.0)
    out_ref[0, 0] = h[:, :32]
    out_ref[0, 1] = h[:, 32:]


def _compute_h(d_pad, cf_W1, cf_b1, cf_W2, cf_b2):
    return pl.pallas_call(
        _h_body,
        grid=(N_CONV, EP // BE),
        in_specs=[
            pl.BlockSpec((BE, 1), lambda i, e: (e, 0)),
            pl.BlockSpec((1, N_CENTERS, DIM), lambda i, e: (i, 0, 0)),
            pl.BlockSpec((1, 1, DIM), lambda i, e: (i, 0, 0)),
            pl.BlockSpec((1, DIM, DIM), lambda i, e: (i, 0, 0)),
            pl.BlockSpec((1, 1, DIM), lambda i, e: (i, 0, 0)),
        ],
        out_specs=pl.BlockSpec((1, 2, BE, 32), lambda i, e: (i, 0, e, 0)),
        out_shape=jax.ShapeDtypeStruct((N_CONV, 2, EP, 32), jnp.float32),
        interpret=_INTERPRET,
    )(d_pad, cf_W1, cf_b1, cf_W2, cf_b2)


BN = 512


def _embed_body(nt_ref, emb_ref, w_ref, node_ref, nn_ref):
    nt = nt_ref[...]
    oh = (nt == lax.broadcasted_iota(jnp.int32, (BN, TYPE_NUM), 1)).astype(jnp.float32)
    nodev = jnp.dot(oh, emb_ref[...], preferred_element_type=jnp.float32)
    node_ref[...] = nodev
    nn = jnp.dot(nodev, w_ref[...], preferred_element_type=jnp.float32)
    nn_ref[0] = nn[:, :32]
    nn_ref[1] = nn[:, 32:]


def _embed(nt_pad, emb, W1_0):
    return pl.pallas_call(
        _embed_body,
        grid=(NP // BN,),
        in_specs=[
            pl.BlockSpec((BN, 1), lambda n: (n, 0)),
            pl.BlockSpec((TYPE_NUM, DIM), lambda n: (0, 0)),
            pl.BlockSpec((DIM, DIM), lambda n: (0, 0)),
        ],
        out_specs=[
            pl.BlockSpec((BN, DIM), lambda n: (n, 0)),
            pl.BlockSpec((2, BN, 32), lambda n: (0, n, 0)),
        ],
        out_shape=[
            jax.ShapeDtypeStruct((NP, DIM), jnp.float32),
            jax.ShapeDtypeStruct((2, NP, 32), jnp.float32),
        ],
        interpret=_INTERPRET,
    )(nt_pad, emb, W1_0)


def _update_body(agg_ref, node_ref, w2_ref, b2_ref, w3_ref, b3_ref, w1n_ref,
                 node_out_ref, nn_ref):
    aggc = jnp.concatenate([agg_ref[0], agg_ref[1]], axis=1)
    cf1 = jnp.dot(aggc, w2_ref[...], preferred_element_type=jnp.float32) + b2_ref[...]
    a = _softplus_b05(cf1)
    upd = jnp.dot(a, w3_ref[...], preferred_element_type=jnp.float32) + b3_ref[...]
    nodev = node_ref[...] + upd
    node_out_ref[...] = nodev
    nn = jnp.dot(nodev, w1n_ref[...], preferred_element_type=jnp.float32)
    nn_ref[0] = nn[:, :32]
    nn_ref[1] = nn[:, 32:]


def _update(agg, node, W2, b2, W3, b3, W1n):
    return pl.pallas_call(
        _update_body,
        grid=(NP // BN,),
        in_specs=[
            pl.BlockSpec((2, BN, 32), lambda n: (0, n, 0)),
            pl.BlockSpec((BN, DIM), lambda n: (n, 0)),
            pl.BlockSpec((DIM, DIM), lambda n: (0, 0)),
            pl.BlockSpec((1, DIM), lambda n: (0, 0)),
            pl.BlockSpec((DIM, DIM), lambda n: (0, 0)),
            pl.BlockSpec((1, DIM), lambda n: (0, 0)),
            pl.BlockSpec((DIM, DIM), lambda n: (0, 0)),
        ],
        out_specs=[
            pl.BlockSpec((BN, DIM), lambda n: (n, 0)),
            pl.BlockSpec((2, BN, 32), lambda n: (0, n, 0)),
        ],
        out_shape=[
            jax.ShapeDtypeStruct((NP, DIM), jnp.float32),
            jax.ShapeDtypeStruct((2, NP, 32), jnp.float32),
        ],
        interpret=_INTERPRET,
    )(agg, node, W2, b2, W3, b3, W1n)


BH = 400
REAL_H_BLOCKS = N // BH


def _head_body(node_ref, d1W_ref, d1b_ref, d2W_ref, d2b_ref, clsW_ref, clsb_ref,
               atomW_ref, atomb_ref, propW_ref, propb_ref,
               atoms_ref, cls_ref, prop_ref, acc_ref):
    n = pl.program_id(0)
    x = node_ref[...]
    a1 = jnp.dot(x, d1W_ref[...], preferred_element_type=jnp.float32) + d1b_ref[...]
    a1 = jnp.logaddexp(0.0, a1) - jnp.log(2.0)
    res = jnp.dot(a1, d2W_ref[...], preferred_element_type=jnp.float32) + d2b_ref[...]
    atoms_ref[...] = jnp.dot(res, atomW_ref[...], preferred_element_type=jnp.float32) + atomb_ref[...]

    @pl.when(n == 0)
    def _():
        acc_ref[...] = jnp.zeros_like(acc_ref)

    @pl.when(n < REAL_H_BLOCKS)
    def _():
        acc_ref[...] += jnp.sum(res, axis=0, keepdims=True)

    @pl.when(n == (NP // BH) - 1)
    def _():
        m = acc_ref[...] * (1.0 / N)
        cls_ref[...] = jnp.dot(m, clsW_ref[...], preferred_element_type=jnp.float32) + clsb_ref[...]
        prop_ref[...] = jnp.dot(m, propW_ref[...], preferred_element_type=jnp.float32) + propb_ref[...]


def _head(node, d1_W, d1_b, d2_W, d2_b, cls_W, cls_b, atom_W, atom_b, prop_W, prop_b):
    full = lambda a: pl.BlockSpec(a.shape, lambda n: (0,) * a.ndim)
    return pl.pallas_call(
        _head_body,
        grid=(NP // BH,),
        in_specs=[
            pl.BlockSpec((BH, DIM), lambda n: (n, 0)),
            full(d1_W), full(d1_b), full(d2_W), full(d2_b),
            full(cls_W), full(cls_b), full(atom_W), full(atom_b),
            full(prop_W), full(prop_b),
        ],
        out_specs=[
            pl.BlockSpec((BH, TYPE_NUM), lambda n: (n, 0)),
            pl.BlockSpec((1, 2000), lambda n: (0, 0)),
            pl.BlockSpec((1, 30), lambda n: (0, 0)),
        ],
        out_shape=[
            jax.ShapeDtypeStruct((NP, TYPE_NUM), jnp.float32),
            jax.ShapeDtypeStruct((1, 2000), jnp.float32),
            jax.ShapeDtypeStruct((1, 30), jnp.float32),
        ],
        scratch_shapes=[pltpu.VMEM((1, 256), jnp.float32)],
        interpret=_INTERPRET,
    )(node, d1_W, d1_b, d2_W, d2_b, cls_W, cls_b, atom_W, atom_b, prop_W, prop_b)


CH = 128
N_CH = 2
SUP = CH * N_CH
N_TILES = 16
E_PER_TILE = EP // N_TILES
N_SUP = E_PER_TILE // SUP
ROWS_PER_TILE = NP // N_TILES


def _make_edge_kernel(layer):
    mesh = plsc.VectorSubcoreMesh(core_axis_name="c", subcore_axis_name="s",
                                  num_cores=2, num_subcores=N_TILES)

    @functools.partial(
        pl.kernel,
        out_type=jax.ShapeDtypeStruct((2, NP, 32), jnp.float32),
        mesh=mesh,
        scratch_types=[
            pltpu.VMEM((N_CH, CH), jnp.int32),
            pltpu.VMEM((N_CH, CH), jnp.int32),
            pltpu.VMEM((SUP, 32), jnp.float32),
            pltpu.VMEM((SUP, 32), jnp.float32),
            pltpu.VMEM_SHARED((NP, 32), jnp.float32),
            pltpu.SemaphoreType.DMA,
        ],
        compiler_params=pltpu.CompilerParams(use_tc_tiling_on_sc=False),
    )
    def ek(src_hbm, dst_hbm, h_hbm, tab_hbm, out_hbm,
           src_v, dst_v, h_v, g_v, agg_sh, sem):
        c = lax.axis_index("c")
        s = lax.axis_index("s")

        def zb(i, carry):
            g_v[i, pl.ds(0, 16)] = jnp.zeros((16,), jnp.float32)
            g_v[i, pl.ds(16, 16)] = jnp.zeros((16,), jnp.float32)
            return carry
        lax.fori_loop(0, CH, zb, 0)
        r_base = pl.multiple_of(s * ROWS_PER_TILE, CH)
        for t in range(ROWS_PER_TILE // CH):
            pltpu.sync_copy(g_v.at[pl.ds(0, CH)],
                            agg_sh.at[pl.ds(r_base + t * CH, CH)])
        plsc.subcore_barrier()

        e_base = s * E_PER_TILE

        def sup_body(g, carry):
            e0 = pl.multiple_of(e_base + g * SUP, SUP)
            row0 = pl.multiple_of(e0 // CH, N_CH)
            pltpu.sync_copy(src_hbm.at[pl.ds(row0, N_CH)], src_v)
            pltpu.sync_copy(dst_hbm.at[pl.ds(row0, N_CH)], dst_v)
            pltpu.sync_copy(h_hbm.at[layer, c, pl.ds(e0, SUP)], h_v)
            descs = [
                pltpu.async_copy(tab_hbm.at[c].at[src_v.at[j]],
                                 g_v.at[pl.ds(j * CH, CH)], sem)
                for j in range(N_CH)
            ]
            for d in descs:
                d.wait()

            def mrow(r, carry2):
                for u in range(4):
                    rr = r * 4 + u
                    for hf in (0, 16):
                        g_v[rr, pl.ds(hf, 16)] = (
                            g_v[rr, pl.ds(hf, 16)] * h_v[rr, pl.ds(hf, 16)]
                        )
                return carry2
            lax.fori_loop(0, SUP // 4, mrow, 0)

            for j in range(N_CH):
                pltpu.sync_copy(g_v.at[pl.ds(j * CH, CH)],
                                agg_sh.at[dst_v.at[j]], add=True)
            return carry
        lax.fori_loop(0, N_SUP, sup_body, 0)

        plsc.subcore_barrier()
        o_base = pl.multiple_of(s * ROWS_PER_TILE, ROWS_PER_TILE)
        pltpu.sync_copy(agg_sh.at[pl.ds(o_base, ROWS_PER_TILE)],
                        out_hbm.at[c, pl.ds(o_base, ROWS_PER_TILE)])

    return ek


_edge_kernel_cache = {}


def _edge_call(layer, src2d, dst2d, h_all, nn):
    if layer not in _edge_kernel_cache:
        _edge_kernel_cache[layer] = _make_edge_kernel(layer)
    return _edge_kernel_cache[layer](src2d, dst2d, h_all, nn)


def kernel(node_type, edge_index, distance, emb, conv_W1, cf_W1, cf_b1, cf_W2,
           cf_b2, conv_W2, conv_b2, conv_W3, conv_b3, d1_W, d1_b, d2_W, d2_b,
           cls_W, cls_b, atom_W, atom_b, prop_W, prop_b):
    i32 = jnp.int32
    nt_pad = jnp.concatenate(
        [node_type.astype(i32), jnp.zeros((NP - N,), i32)]).reshape(NP, 1)
    src2d = jnp.concatenate(
        [edge_index[0].astype(i32), jnp.zeros((EP - E,), i32)]).reshape(EP // CH, CH)
    dst2d = jnp.concatenate(
        [edge_index[1].astype(i32), jnp.zeros((EP - E,), i32)]).reshape(EP // CH, CH)
    d_pad = jnp.concatenate(
        [distance, jnp.zeros((EP - E,), jnp.float32)]).reshape(EP, 1)

    h_all = _compute_h(d_pad, cf_W1, cf_b1.reshape(N_CONV, 1, DIM), cf_W2,
                       cf_b2.reshape(N_CONV, 1, DIM))
    node, nn = _embed(nt_pad, emb, conv_W1[0])

    for i in range(N_CONV):
        agg = _edge_call(i, src2d, dst2d, h_all, nn)
        W1n = conv_W1[i + 1] if i + 1 < N_CONV else conv_W1[0]
        node, nn = _update(agg, node, conv_W2[i], conv_b2[i].reshape(1, DIM),
                           conv_W3[i], conv_b3[i].reshape(1, DIM), W1n)

    atoms, cls_p, prop_p = _head(node, d1_W, d1_b.reshape(1, 256), d2_W,
                                 d2_b.reshape(1, 256), cls_W, cls_b.reshape(1, 2000),
                                 atom_W, atom_b.reshape(1, TYPE_NUM), prop_W,
                                 prop_b.reshape(1, 30))
    return (atoms[:N], cls_p, prop_p)


# speedup vs baseline: 1.5687x; 1.1779x over previous
import functools

import numpy as np

import jax
import jax.numpy as jnp
from jax import lax
from jax.experimental import pallas as pl
from jax.experimental.pallas import tpu as pltpu
from jax.experimental.pallas import tpu_sc as plsc

N = 50000
E = 800000
DIM = 64
TYPE_NUM = 100
N_CONV = 3
CUTOFF = 5.0
N_CENTERS = 5
GAP = CUTOFF / (N_CENTERS - 1)
_CENTERS_NP = np.linspace(0.0, CUTOFF, N_CENTERS).astype(np.float32)

NP = 51200
EP = 819200

_INTERPRET = False

BE = 1024
HG = 256


def _softplus_b05(x):
    return 2.0 * jnp.logaddexp(0.0, 0.5 * x)


def _h_body(d_ref, w1_ref, b1_ref, w2_ref, b2_ref, out_ref):
    e = pl.program_id(1)
    d = d_ref[...]
    rbf = jnp.concatenate(
        [jnp.exp((-1.0 / GAP) * (d - float(c)) ** 2) for c in _CENTERS_NP],
        axis=1)
    t = jnp.dot(rbf, w1_ref[0], preferred_element_type=jnp.float32) + b1_ref[0]
    t = _softplus_b05(t)
    h = jnp.dot(t, w2_ref[0], preferred_element_type=jnp.float32) + b2_ref[0]
    ids = e * BE + lax.broadcasted_iota(jnp.int32, (BE, 1), 0)
    h = jnp.where(ids < E, h, 0.0)
    for cc in range(2):
        cols = h[:, cc * 32:(cc + 1) * 32]
        out_ref[0, cc] = jnp.concatenate(
            [cols[HG * u:HG * (u + 1)] for u in range(BE // HG)], axis=1)


def _compute_h(d_pad, cf_W1, cf_b1, cf_W2, cf_b2):
    return pl.pallas_call(
        _h_body,
        grid=(N_CONV, EP // BE),
        in_specs=[
            pl.BlockSpec((BE, 1), lambda i, e: (e, 0)),
            pl.BlockSpec((1, N_CENTERS, DIM), lambda i, e: (i, 0, 0)),
            pl.BlockSpec((1, 1, DIM), lambda i, e: (i, 0, 0)),
            pl.BlockSpec((1, DIM, DIM), lambda i, e: (i, 0, 0)),
            pl.BlockSpec((1, 1, DIM), lambda i, e: (i, 0, 0)),
        ],
        out_specs=pl.BlockSpec((1, 2, HG, 128), lambda i, e: (i, 0, e, 0)),
        out_shape=jax.ShapeDtypeStruct((N_CONV, 2, EP // 4, 128), jnp.float32),
        name="h_filter",
        interpret=_INTERPRET,
    )(d_pad, cf_W1, cf_b1, cf_W2, cf_b2)


BN = 512


def _embed_body(nt_ref, emb_ref, w_ref, node_ref, nn_ref):
    nt = nt_ref[...]
    oh = (nt == lax.broadcasted_iota(jnp.int32, (BN, TYPE_NUM), 1)).astype(jnp.float32)
    nodev = jnp.dot(oh, emb_ref[...], preferred_element_type=jnp.float32)
    node_ref[...] = nodev
    nn = jnp.dot(nodev, w_ref[...], preferred_element_type=jnp.float32)
    nn_ref[0] = nn[:, :32]
    nn_ref[1] = nn[:, 32:]


def _embed(nt_pad, emb, W1_0):
    return pl.pallas_call(
        _embed_body,
        grid=(NP // BN,),
        in_specs=[
            pl.BlockSpec((BN, 1), lambda n: (n, 0)),
            pl.BlockSpec((TYPE_NUM, DIM), lambda n: (0, 0)),
            pl.BlockSpec((DIM, DIM), lambda n: (0, 0)),
        ],
        out_specs=[
            pl.BlockSpec((BN, DIM), lambda n: (n, 0)),
            pl.BlockSpec((2, BN, 32), lambda n: (0, n, 0)),
        ],
        out_shape=[
            jax.ShapeDtypeStruct((NP, DIM), jnp.float32),
            jax.ShapeDtypeStruct((2, NP, 32), jnp.float32),
        ],
        name="embed_w1",
        interpret=_INTERPRET,
    )(nt_pad, emb, W1_0)


def _update_body(agg_ref, node_ref, w2_ref, b2_ref, w3_ref, b3_ref, w1n_ref,
                 node_out_ref, nn_ref):
    aggc = jnp.concatenate([agg_ref[0], agg_ref[1]], axis=1)
    cf1 = jnp.dot(aggc, w2_ref[...], preferred_element_type=jnp.float32) + b2_ref[...]
    a = _softplus_b05(cf1)
    upd = jnp.dot(a, w3_ref[...], preferred_element_type=jnp.float32) + b3_ref[...]
    nodev = node_ref[...] + upd
    node_out_ref[...] = nodev
    nn = jnp.dot(nodev, w1n_ref[...], preferred_element_type=jnp.float32)
    nn_ref[0] = nn[:, :32]
    nn_ref[1] = nn[:, 32:]


def _update(agg, node, W2, b2, W3, b3, W1n):
    return pl.pallas_call(
        _update_body,
        grid=(NP // BN,),
        in_specs=[
            pl.BlockSpec((2, BN, 32), lambda n: (0, n, 0)),
            pl.BlockSpec((BN, DIM), lambda n: (n, 0)),
            pl.BlockSpec((DIM, DIM), lambda n: (0, 0)),
            pl.BlockSpec((1, DIM), lambda n: (0, 0)),
            pl.BlockSpec((DIM, DIM), lambda n: (0, 0)),
            pl.BlockSpec((1, DIM), lambda n: (0, 0)),
            pl.BlockSpec((DIM, DIM), lambda n: (0, 0)),
        ],
        out_specs=[
            pl.BlockSpec((BN, DIM), lambda n: (n, 0)),
            pl.BlockSpec((2, BN, 32), lambda n: (0, n, 0)),
        ],
        out_shape=[
            jax.ShapeDtypeStruct((NP, DIM), jnp.float32),
            jax.ShapeDtypeStruct((2, NP, 32), jnp.float32),
        ],
        name="node_update",
        interpret=_INTERPRET,
    )(agg, node, W2, b2, W3, b3, W1n)


BH = 400
REAL_H_BLOCKS = N // BH


def _head_body(node_ref, d1W_ref, d1b_ref, d2W_ref, d2b_ref, clsW_ref, clsb_ref,
               atomW_ref, atomb_ref, propW_ref, propb_ref,
               atoms_ref, cls_ref, prop_ref, acc_ref):
    n = pl.program_id(0)
    x = node_ref[...]
    a1 = jnp.dot(x, d1W_ref[...], preferred_element_type=jnp.float32) + d1b_ref[...]
    a1 = jnp.logaddexp(0.0, a1) - jnp.log(2.0)
    res = jnp.dot(a1, d2W_ref[...], preferred_element_type=jnp.float32) + d2b_ref[...]
    atoms_ref[...] = jnp.dot(res, atomW_ref[...], preferred_element_type=jnp.float32) + atomb_ref[...]

    @pl.when(n == 0)
    def _():
        acc_ref[...] = jnp.zeros_like(acc_ref)

    @pl.when(n < REAL_H_BLOCKS)
    def _():
        acc_ref[...] += jnp.sum(res, axis=0, keepdims=True)

    @pl.when(n == (NP // BH) - 1)
    def _():
        m = acc_ref[...] * (1.0 / N)
        cls_ref[...] = jnp.dot(m, clsW_ref[...], preferred_element_type=jnp.float32) + clsb_ref[...]
        prop_ref[...] = jnp.dot(m, propW_ref[...], preferred_element_type=jnp.float32) + propb_ref[...]


def _head(node, d1_W, d1_b, d2_W, d2_b, cls_W, cls_b, atom_W, atom_b, prop_W, prop_b):
    full = lambda a: pl.BlockSpec(a.shape, lambda n: (0,) * a.ndim)
    return pl.pallas_call(
        _head_body,
        grid=(NP // BH,),
        in_specs=[
            pl.BlockSpec((BH, DIM), lambda n: (n, 0)),
            full(d1_W), full(d1_b), full(d2_W), full(d2_b),
            full(cls_W), full(cls_b), full(atom_W), full(atom_b),
            full(prop_W), full(prop_b),
        ],
        out_specs=[
            pl.BlockSpec((BH, TYPE_NUM), lambda n: (n, 0)),
            pl.BlockSpec((1, 2000), lambda n: (0, 0)),
            pl.BlockSpec((1, 30), lambda n: (0, 0)),
        ],
        out_shape=[
            jax.ShapeDtypeStruct((NP, TYPE_NUM), jnp.float32),
            jax.ShapeDtypeStruct((1, 2000), jnp.float32),
            jax.ShapeDtypeStruct((1, 30), jnp.float32),
        ],
        scratch_shapes=[pltpu.VMEM((1, 256), jnp.float32)],
        name="mlp_head",
        interpret=_INTERPRET,
    )(node, d1_W, d1_b, d2_W, d2_b, cls_W, cls_b, atom_W, atom_b, prop_W, prop_b)


CH = 128
N_CH = 2
SUP = CH * N_CH
N_TILES = 16
E_PER_TILE = EP // N_TILES
N_SUP = E_PER_TILE // SUP
ROWS_PER_TILE = NP // N_TILES


def _make_edge_kernel(layer):
    mesh = plsc.VectorSubcoreMesh(core_axis_name="c", subcore_axis_name="s",
                                  num_cores=2, num_subcores=N_TILES)

    @functools.partial(
        pl.kernel,
        out_type=jax.ShapeDtypeStruct((2, NP, 32), jnp.float32),
        mesh=mesh,
        scratch_types=[
            pltpu.VMEM((N_CH, CH), jnp.int32),
            pltpu.VMEM((N_CH, CH), jnp.int32),
            pltpu.VMEM((SUP, 32), jnp.float32),
            pltpu.VMEM((SUP, 32), jnp.float32),
            pltpu.VMEM_SHARED((NP, 32), jnp.float32),
            pltpu.SemaphoreType.DMA,
        ],
        compiler_params=pltpu.CompilerParams(use_tc_tiling_on_sc=False),
        name=f"sc_edge{layer}",
    )
    def ek(src_hbm, dst_hbm, h_hbm, tab_hbm, out_hbm,
           src_v, dst_v, h_v, g_v, agg_sh, sem):
        c = lax.axis_index("c")
        s = lax.axis_index("s")

        def zb(i, carry):
            g_v[i, pl.ds(0, 16)] = jnp.zeros((16,), jnp.float32)
            g_v[i, pl.ds(16, 16)] = jnp.zeros((16,), jnp.float32)
            return carry
        lax.fori_loop(0, CH, zb, 0)
        r_base = pl.multiple_of(s * ROWS_PER_TILE, CH)
        for t in range(ROWS_PER_TILE // CH):
            pltpu.sync_copy(g_v.at[pl.ds(0, CH)],
                            agg_sh.at[pl.ds(r_base + t * CH, CH)])
        plsc.subcore_barrier()

        e_base = s * E_PER_TILE

        def sup_body(g, carry):
            e0 = pl.multiple_of(e_base + g * SUP, SUP)
            row0 = pl.multiple_of(e0 // CH, N_CH)
            pltpu.sync_copy(src_hbm.at[pl.ds(row0, N_CH)], src_v)
            pltpu.sync_copy(dst_hbm.at[pl.ds(row0, N_CH)], dst_v)
            q_row = pl.multiple_of((e0 // BE) * HG, HG)
            q_lane = pl.multiple_of(((e0 // SUP) % (BE // HG)) * 32, 32)
            pltpu.sync_copy(h_hbm.at[layer, c, pl.ds(q_row, SUP), pl.ds(q_lane, 32)],
                            h_v)
            descs = [
                pltpu.async_copy(tab_hbm.at[c].at[src_v.at[j]],
                                 g_v.at[pl.ds(j * CH, CH)], sem)
                for j in range(N_CH)
            ]
            for d in descs:
                d.wait()

            def mrow(r, carry2):
                for u in range(4):
                    rr = r * 4 + u
                    for hf in (0, 16):
                        g_v[rr, pl.ds(hf, 16)] = (
                            g_v[rr, pl.ds(hf, 16)] * h_v[rr, pl.ds(hf, 16)]
                        )
                return carry2
            lax.fori_loop(0, SUP // 4, mrow, 0)

            for j in range(N_CH):
                pltpu.sync_copy(g_v.at[pl.ds(j * CH, CH)],
                                agg_sh.at[dst_v.at[j]], add=True)
            return carry
        lax.fori_loop(0, N_SUP, sup_body, 0)

        plsc.subcore_barrier()
        o_base = pl.multiple_of(s * ROWS_PER_TILE, ROWS_PER_TILE)
        pltpu.sync_copy(agg_sh.at[pl.ds(o_base, ROWS_PER_TILE)],
                        out_hbm.at[c, pl.ds(o_base, ROWS_PER_TILE)])

    return ek


_edge_kernel_cache = {}


def _edge_call(layer, src2d, dst2d, h_all, nn):
    if layer not in _edge_kernel_cache:
        _edge_kernel_cache[layer] = _make_edge_kernel(layer)
    return _edge_kernel_cache[layer](src2d, dst2d, h_all, nn)


def kernel(node_type, edge_index, distance, emb, conv_W1, cf_W1, cf_b1, cf_W2,
           cf_b2, conv_W2, conv_b2, conv_W3, conv_b3, d1_W, d1_b, d2_W, d2_b,
           cls_W, cls_b, atom_W, atom_b, prop_W, prop_b):
    i32 = jnp.int32
    nt_pad = jnp.concatenate(
        [node_type.astype(i32), jnp.zeros((NP - N,), i32)]).reshape(NP, 1)
    src2d = jnp.concatenate(
        [edge_index[0].astype(i32), jnp.zeros((EP - E,), i32)]).reshape(EP // CH, CH)
    dst2d = jnp.concatenate(
        [edge_index[1].astype(i32), jnp.zeros((EP - E,), i32)]).reshape(EP // CH, CH)
    d_pad = jnp.concatenate(
        [distance, jnp.zeros((EP - E,), jnp.float32)]).reshape(EP, 1)

    h_all = _compute_h(d_pad, cf_W1, cf_b1.reshape(N_CONV, 1, DIM), cf_W2,
                       cf_b2.reshape(N_CONV, 1, DIM))
    node, nn = _embed(nt_pad, emb, conv_W1[0])

    for i in range(N_CONV):
        agg = _edge_call(i, src2d, dst2d, h_all, nn)
        W1n = conv_W1[i + 1] if i + 1 < N_CONV else conv_W1[0]
        node, nn = _update(agg, node, conv_W2[i], conv_b2[i].reshape(1, DIM),
                           conv_W3[i], conv_b3[i].reshape(1, DIM), W1n)

    atoms, cls_p, prop_p = _head(node, d1_W, d1_b.reshape(1, 256), d2_W,
                                 d2_b.reshape(1, 256), cls_W, cls_b.reshape(1, 2000),
                                 atom_W, atom_b.reshape(1, TYPE_NUM), prop_W,
                                 prop_b.reshape(1, 30))
    return (atoms[:N], cls_p, prop_p)


# speedup vs baseline: 1.9335x; 1.2325x over previous
import functools

import numpy as np

import jax
import jax.numpy as jnp
from jax import lax
from jax.experimental import pallas as pl
from jax.experimental.pallas import tpu as pltpu
from jax.experimental.pallas import tpu_sc as plsc

N = 50000
E = 800000
DIM = 64
TYPE_NUM = 100
N_CONV = 3
CUTOFF = 5.0
N_CENTERS = 5
GAP = CUTOFF / (N_CENTERS - 1)
_CENTERS_NP = np.linspace(0.0, CUTOFF, N_CENTERS).astype(np.float32)

NP = 51200
EP = 819200

_INTERPRET = False

EB = 8192


def _softplus_b05(x):
    return 2.0 * jnp.logaddexp(0.0, 0.5 * x)


def _h_body(d_ref, w1t_ref, b1_ref, w2t_ref, b2_ref, out_ref):
    e = pl.program_id(1)
    d = d_ref[...]
    rbf_t = jnp.concatenate(
        [jnp.exp((-1.0 / GAP) * (d - float(c)) ** 2) for c in _CENTERS_NP],
        axis=0)
    t = jnp.dot(w1t_ref[0], rbf_t, preferred_element_type=jnp.float32) + b1_ref[0]
    t = _softplus_b05(t)
    h = jnp.dot(w2t_ref[0], t, preferred_element_type=jnp.float32) + b2_ref[0]
    ids = e * EB + lax.broadcasted_iota(jnp.int32, (1, EB), 1)
    out_ref[0] = jnp.where(ids < E, h, 0.0)


def _compute_h(d_pad, cf_W1T, cf_b1, cf_W2T, cf_b2):
    return pl.pallas_call(
        _h_body,
        grid=(N_CONV, EP // EB),
        in_specs=[
            pl.BlockSpec((1, EB), lambda i, e: (0, e)),
            pl.BlockSpec((1, DIM, N_CENTERS), lambda i, e: (i, 0, 0)),
            pl.BlockSpec((1, DIM, 1), lambda i, e: (i, 0, 0)),
            pl.BlockSpec((1, DIM, DIM), lambda i, e: (i, 0, 0)),
            pl.BlockSpec((1, DIM, 1), lambda i, e: (i, 0, 0)),
        ],
        out_specs=pl.BlockSpec((1, DIM, EB), lambda i, e: (i, 0, e)),
        out_shape=jax.ShapeDtypeStruct((N_CONV, DIM, EP), jnp.float32),
        name="h_filter",
        interpret=_INTERPRET,
    )(d_pad, cf_W1T, cf_b1, cf_W2T, cf_b2)


BN = 512


def _embed_body(nt_ref, emb_ref, w_ref, node_ref, nn_ref):
    nt = nt_ref[...]
    oh = (nt == lax.broadcasted_iota(jnp.int32, (BN, TYPE_NUM), 1)).astype(jnp.float32)
    nodev = jnp.dot(oh, emb_ref[...], preferred_element_type=jnp.float32)
    node_ref[...] = nodev
    nn = jnp.dot(nodev, w_ref[...], preferred_element_type=jnp.float32)
    nn_ref[0] = nn[:, :32]
    nn_ref[1] = nn[:, 32:]


def _embed(nt_pad, emb, W1_0):
    return pl.pallas_call(
        _embed_body,
        grid=(NP // BN,),
        in_specs=[
            pl.BlockSpec((BN, 1), lambda n: (n, 0)),
            pl.BlockSpec((TYPE_NUM, DIM), lambda n: (0, 0)),
            pl.BlockSpec((DIM, DIM), lambda n: (0, 0)),
        ],
        out_specs=[
            pl.BlockSpec((BN, DIM), lambda n: (n, 0)),
            pl.BlockSpec((2, BN, 32), lambda n: (0, n, 0)),
        ],
        out_shape=[
            jax.ShapeDtypeStruct((NP, DIM), jnp.float32),
            jax.ShapeDtypeStruct((2, NP, 32), jnp.float32),
        ],
        name="embed_w1",
        interpret=_INTERPRET,
    )(nt_pad, emb, W1_0)


def _update_body(agg_ref, node_ref, w2_ref, b2_ref, w3_ref, b3_ref, w1n_ref,
                 node_out_ref, nn_ref):
    aggc = jnp.concatenate([agg_ref[0], agg_ref[1]], axis=1)
    cf1 = jnp.dot(aggc, w2_ref[...], preferred_element_type=jnp.float32) + b2_ref[...]
    a = _softplus_b05(cf1)
    upd = jnp.dot(a, w3_ref[...], preferred_element_type=jnp.float32) + b3_ref[...]
    nodev = node_ref[...] + upd
    node_out_ref[...] = nodev
    nn = jnp.dot(nodev, w1n_ref[...], preferred_element_type=jnp.float32)
    nn_ref[0] = nn[:, :32]
    nn_ref[1] = nn[:, 32:]


def _update(agg, node, W2, b2, W3, b3, W1n):
    return pl.pallas_call(
        _update_body,
        grid=(NP // BN,),
        in_specs=[
            pl.BlockSpec((2, BN, 32), lambda n: (0, n, 0)),
            pl.BlockSpec((BN, DIM), lambda n: (n, 0)),
            pl.BlockSpec((DIM, DIM), lambda n: (0, 0)),
            pl.BlockSpec((1, DIM), lambda n: (0, 0)),
            pl.BlockSpec((DIM, DIM), lambda n: (0, 0)),
            pl.BlockSpec((1, DIM), lambda n: (0, 0)),
            pl.BlockSpec((DIM, DIM), lambda n: (0, 0)),
        ],
        out_specs=[
            pl.BlockSpec((BN, DIM), lambda n: (n, 0)),
            pl.BlockSpec((2, BN, 32), lambda n: (0, n, 0)),
        ],
        out_shape=[
            jax.ShapeDtypeStruct((NP, DIM), jnp.float32),
            jax.ShapeDtypeStruct((2, NP, 32), jnp.float32),
        ],
        name="node_update",
        interpret=_INTERPRET,
    )(agg, node, W2, b2, W3, b3, W1n)


BH = 400
REAL_H_BLOCKS = N // BH


def _head_body(node_ref, d1W_ref, d1b_ref, d2W_ref, d2b_ref, clsW_ref, clsb_ref,
               atomW_ref, atomb_ref, propW_ref, propb_ref,
               atoms_ref, cls_ref, prop_ref, acc_ref):
    n = pl.program_id(0)
    x = node_ref[...]
    a1 = jnp.dot(x, d1W_ref[...], preferred_element_type=jnp.float32) + d1b_ref[...]
    a1 = jnp.logaddexp(0.0, a1) - jnp.log(2.0)
    res = jnp.dot(a1, d2W_ref[...], preferred_element_type=jnp.float32) + d2b_ref[...]
    atoms_ref[...] = jnp.dot(res, atomW_ref[...], preferred_element_type=jnp.float32) + atomb_ref[...]

    @pl.when(n == 0)
    def _():
        acc_ref[...] = jnp.zeros_like(acc_ref)

    @pl.when(n < REAL_H_BLOCKS)
    def _():
        acc_ref[...] += jnp.sum(res, axis=0, keepdims=True)

    @pl.when(n == (NP // BH) - 1)
    def _():
        m = acc_ref[...] * (1.0 / N)
        cls_ref[...] = jnp.dot(m, clsW_ref[...], preferred_element_type=jnp.float32) + clsb_ref[...]
        prop_ref[...] = jnp.dot(m, propW_ref[...], preferred_element_type=jnp.float32) + propb_ref[...]


def _head(node, d1_W, d1_b, d2_W, d2_b, cls_W, cls_b, atom_W, atom_b, prop_W, prop_b):
    full = lambda a: pl.BlockSpec(a.shape, lambda n: (0,) * a.ndim)
    return pl.pallas_call(
        _head_body,
        grid=(NP // BH,),
        in_specs=[
            pl.BlockSpec((BH, DIM), lambda n: (n, 0)),
            full(d1_W), full(d1_b), full(d2_W), full(d2_b),
            full(cls_W), full(cls_b), full(atom_W), full(atom_b),
            full(prop_W), full(prop_b),
        ],
        out_specs=[
            pl.BlockSpec((BH, TYPE_NUM), lambda n: (n, 0)),
            pl.BlockSpec((1, 2000), lambda n: (0, 0)),
            pl.BlockSpec((1, 30), lambda n: (0, 0)),
        ],
        out_shape=[
            jax.ShapeDtypeStruct((NP, TYPE_NUM), jnp.float32),
            jax.ShapeDtypeStruct((1, 2000), jnp.float32),
            jax.ShapeDtypeStruct((1, 30), jnp.float32),
        ],
        scratch_shapes=[pltpu.VMEM((1, 256), jnp.float32)],
        name="mlp_head",
        interpret=_INTERPRET,
    )(node, d1_W, d1_b, d2_W, d2_b, cls_W, cls_b, atom_W, atom_b, prop_W, prop_b)


CH = 128
N_CH = 2
SUP = CH * N_CH
N_TILES = 16
E_PER_TILE = EP // N_TILES
N_SUP = E_PER_TILE // SUP
ROWS_PER_TILE = NP // N_TILES


def _make_edge_kernel(layer):
    mesh = plsc.VectorSubcoreMesh(core_axis_name="c", subcore_axis_name="s",
                                  num_cores=2, num_subcores=N_TILES)

    @functools.partial(
        pl.kernel,
        out_type=jax.ShapeDtypeStruct((2, NP, 32), jnp.float32),
        mesh=mesh,
        scratch_types=[
            pltpu.VMEM((N_CH, CH), jnp.int32),
            pltpu.VMEM((N_CH, CH), jnp.int32),
            pltpu.VMEM((32, SUP + 1), jnp.float32),
            pltpu.VMEM((SUP, 32), jnp.float32),
            pltpu.VMEM_SHARED((NP, 32), jnp.float32),
            pltpu.SemaphoreType.DMA,
        ],
        compiler_params=pltpu.CompilerParams(use_tc_tiling_on_sc=False,
                                             needs_layout_passes=False),
        name=f"sc_edge{layer}",
    )
    def ek(src_hbm, dst_hbm, h_hbm, tab_hbm, out_hbm,
           src_v, dst_v, h_v, g_v, agg_sh, sem):
        c = lax.axis_index("c")
        s = lax.axis_index("s")

        def zb(i, carry):
            g_v[i, pl.ds(0, 16)] = jnp.zeros((16,), jnp.float32)
            g_v[i, pl.ds(16, 16)] = jnp.zeros((16,), jnp.float32)
            return carry
        lax.fori_loop(0, CH, zb, 0)
        r_base = pl.multiple_of(s * ROWS_PER_TILE, CH)
        for t in range(ROWS_PER_TILE // CH):
            pltpu.sync_copy(g_v.at[pl.ds(0, CH)],
                            agg_sh.at[pl.ds(r_base + t * CH, CH)])
        plsc.subcore_barrier()

        e_base = s * E_PER_TILE

        def sup_body(g, carry):
            e0 = pl.multiple_of(e_base + g * SUP, SUP)
            row0 = pl.multiple_of(e0 // CH, N_CH)
            pltpu.sync_copy(src_hbm.at[pl.ds(row0, N_CH)], src_v)
            pltpu.sync_copy(dst_hbm.at[pl.ds(row0, N_CH)], dst_v)
            f0 = pl.multiple_of(c * 32, 32)
            pltpu.sync_copy(h_hbm.at[layer, pl.ds(f0, 32), pl.ds(e0, SUP)],
                            h_v.at[:, pl.ds(0, SUP)])
            descs = [
                pltpu.async_copy(tab_hbm.at[c].at[src_v.at[j]],
                                 g_v.at[pl.ds(j * CH, CH)], sem)
                for j in range(N_CH)
            ]
            for d in descs:
                d.wait()

            iota16 = lax.iota(jnp.int32, 16)
            iota16b = iota16 + 16

            def mrow(r, carry2):
                for u in range(4):
                    rr = r * 4 + u
                    ev = jnp.zeros((16,), jnp.int32) + rr
                    h0 = plsc.load_gather(h_v, [iota16, ev])
                    h1 = plsc.load_gather(h_v, [iota16b, ev])
                    g_v[rr, pl.ds(0, 16)] = g_v[rr, pl.ds(0, 16)] * h0
                    g_v[rr, pl.ds(16, 16)] = g_v[rr, pl.ds(16, 16)] * h1
                return carry2
            lax.fori_loop(0, SUP // 4, mrow, 0)

            for j in range(N_CH):
                pltpu.sync_copy(g_v.at[pl.ds(j * CH, CH)],
                                agg_sh.at[dst_v.at[j]], add=True)
            return carry
        lax.fori_loop(0, N_SUP, sup_body, 0)

        plsc.subcore_barrier()
        o_base = pl.multiple_of(s * ROWS_PER_TILE, ROWS_PER_TILE)
        pltpu.sync_copy(agg_sh.at[pl.ds(o_base, ROWS_PER_TILE)],
                        out_hbm.at[c, pl.ds(o_base, ROWS_PER_TILE)])

    return ek


_edge_kernel_cache = {}


def _edge_call(layer, src2d, dst2d, h_all, nn):
    if layer not in _edge_kernel_cache:
        _edge_kernel_cache[layer] = _make_edge_kernel(layer)
    return _edge_kernel_cache[layer](src2d, dst2d, h_all, nn)


def kernel(node_type, edge_index, distance, emb, conv_W1, cf_W1, cf_b1, cf_W2,
           cf_b2, conv_W2, conv_b2, conv_W3, conv_b3, d1_W, d1_b, d2_W, d2_b,
           cls_W, cls_b, atom_W, atom_b, prop_W, prop_b):
    i32 = jnp.int32
    nt_pad = jnp.concatenate(
        [node_type.astype(i32), jnp.zeros((NP - N,), i32)]).reshape(NP, 1)
    src2d = jnp.concatenate(
        [edge_index[0].astype(i32), jnp.zeros((EP - E,), i32)]).reshape(EP // CH, CH)
    dst2d = jnp.concatenate(
        [edge_index[1].astype(i32), jnp.zeros((EP - E,), i32)]).reshape(EP // CH, CH)
    d_pad = jnp.concatenate(
        [distance, jnp.zeros((EP - E,), jnp.float32)]).reshape(1, EP)

    h_all = _compute_h(d_pad, jnp.swapaxes(cf_W1, 1, 2),
                       cf_b1.reshape(N_CONV, DIM, 1),
                       jnp.swapaxes(cf_W2, 1, 2),
                       cf_b2.reshape(N_CONV, DIM, 1))
    node, nn = _embed(nt_pad, emb, conv_W1[0])

    for i in range(N_CONV):
        agg = _edge_call(i, src2d, dst2d, h_all, nn)
        W1n = conv_W1[i + 1] if i + 1 < N_CONV else conv_W1[0]
        node, nn = _update(agg, node, conv_W2[i], conv_b2[i].reshape(1, DIM),
                           conv_W3[i], conv_b3[i].reshape(1, DIM), W1n)

    atoms, cls_p, prop_p = _head(node, d1_W, d1_b.reshape(1, 256), d2_W,
                                 d2_b.reshape(1, 256), cls_W, cls_b.reshape(1, 2000),
                                 atom_W, atom_b.reshape(1, TYPE_NUM), prop_W,
                                 prop_b.reshape(1, 30))
    return (atoms[:N], cls_p, prop_p)


# speedup vs baseline: 2.0240x; 1.0468x over previous
import functools

import numpy as np

import jax
import jax.numpy as jnp
from jax import lax
from jax.experimental import pallas as pl
from jax.experimental.pallas import tpu as pltpu
from jax.experimental.pallas import tpu_sc as plsc

N = 50000
E = 800000
DIM = 64
TYPE_NUM = 100
N_CONV = 3
CUTOFF = 5.0
N_CENTERS = 5
GAP = CUTOFF / (N_CENTERS - 1)
_CENTERS_NP = np.linspace(0.0, CUTOFF, N_CENTERS).astype(np.float32)

NP = 51200
EP = 819200

_INTERPRET = False

EB = 8192


def _softplus_b05(x):
    return 2.0 * jnp.logaddexp(0.0, 0.5 * x)


def _h_body(d_ref, w1t_ref, b1_ref, w2t_ref, b2_ref, out_ref):
    e = pl.program_id(1)
    d = d_ref[...]
    rbf_t = jnp.concatenate(
        [jnp.exp((-1.0 / GAP) * (d - float(c)) ** 2) for c in _CENTERS_NP],
        axis=0)
    t = jnp.dot(w1t_ref[0], rbf_t, preferred_element_type=jnp.float32) + b1_ref[0]
    t = _softplus_b05(t)
    h = jnp.dot(w2t_ref[0], t, preferred_element_type=jnp.float32) + b2_ref[0]
    ids = e * EB + lax.broadcasted_iota(jnp.int32, (1, EB), 1)
    hm = jnp.where(ids < E, h, 0.0)
    for b in range(EB // 128):
        out_ref[0, 0, b] = hm[:, 128 * b:128 * (b + 1)]


def _compute_h(d_pad, cf_W1T, cf_b1, cf_W2T, cf_b2):
    return pl.pallas_call(
        _h_body,
        grid=(N_CONV, EP // EB),
        in_specs=[
            pl.BlockSpec((1, EB), lambda i, e: (0, e)),
            pl.BlockSpec((1, DIM, N_CENTERS), lambda i, e: (i, 0, 0)),
            pl.BlockSpec((1, DIM, 1), lambda i, e: (i, 0, 0)),
            pl.BlockSpec((1, DIM, DIM), lambda i, e: (i, 0, 0)),
            pl.BlockSpec((1, DIM, 1), lambda i, e: (i, 0, 0)),
        ],
        out_specs=pl.BlockSpec((1, 1, EB // 128, DIM, 128),
                               lambda i, e: (i, e, 0, 0, 0)),
        out_shape=jax.ShapeDtypeStruct(
            (N_CONV, EP // EB, EB // 128, DIM, 128), jnp.float32),
        name="h_filter",
        interpret=_INTERPRET,
    )(d_pad, cf_W1T, cf_b1, cf_W2T, cf_b2)


BN = 512


def _embed_body(nt_ref, emb_ref, w_ref, node_ref, nn_ref):
    nt = nt_ref[...]
    oh = (nt == lax.broadcasted_iota(jnp.int32, (BN, TYPE_NUM), 1)).astype(jnp.float32)
    nodev = jnp.dot(oh, emb_ref[...], preferred_element_type=jnp.float32)
    node_ref[...] = nodev
    nn = jnp.dot(nodev, w_ref[...], preferred_element_type=jnp.float32)
    nn_ref[0] = nn[:, :32]
    nn_ref[1] = nn[:, 32:]


def _embed(nt_pad, emb, W1_0):
    return pl.pallas_call(
        _embed_body,
        grid=(NP // BN,),
        in_specs=[
            pl.BlockSpec((BN, 1), lambda n: (n, 0)),
            pl.BlockSpec((TYPE_NUM, DIM), lambda n: (0, 0)),
            pl.BlockSpec((DIM, DIM), lambda n: (0, 0)),
        ],
        out_specs=[
            pl.BlockSpec((BN, DIM), lambda n: (n, 0)),
            pl.BlockSpec((2, BN, 32), lambda n: (0, n, 0)),
        ],
        out_shape=[
            jax.ShapeDtypeStruct((NP, DIM), jnp.float32),
            jax.ShapeDtypeStruct((2, NP, 32), jnp.float32),
        ],
        name="embed_w1",
        interpret=_INTERPRET,
    )(nt_pad, emb, W1_0)


def _update_body(agg_ref, node_ref, w2_ref, b2_ref, w3_ref, b3_ref, w1n_ref,
                 node_out_ref, nn_ref):
    aggc = jnp.concatenate([agg_ref[0], agg_ref[1]], axis=1)
    cf1 = jnp.dot(aggc, w2_ref[...], preferred_element_type=jnp.float32) + b2_ref[...]
    a = _softplus_b05(cf1)
    upd = jnp.dot(a, w3_ref[...], preferred_element_type=jnp.float32) + b3_ref[...]
    nodev = node_ref[...] + upd
    node_out_ref[...] = nodev
    nn = jnp.dot(nodev, w1n_ref[...], preferred_element_type=jnp.float32)
    nn_ref[0] = nn[:, :32]
    nn_ref[1] = nn[:, 32:]


def _update(agg, node, W2, b2, W3, b3, W1n):
    return pl.pallas_call(
        _update_body,
        grid=(NP // BN,),
        in_specs=[
            pl.BlockSpec((2, BN, 32), lambda n: (0, n, 0)),
            pl.BlockSpec((BN, DIM), lambda n: (n, 0)),
            pl.BlockSpec((DIM, DIM), lambda n: (0, 0)),
            pl.BlockSpec((1, DIM), lambda n: (0, 0)),
            pl.BlockSpec((DIM, DIM), lambda n: (0, 0)),
            pl.BlockSpec((1, DIM), lambda n: (0, 0)),
            pl.BlockSpec((DIM, DIM), lambda n: (0, 0)),
        ],
        out_specs=[
            pl.BlockSpec((BN, DIM), lambda n: (n, 0)),
            pl.BlockSpec((2, BN, 32), lambda n: (0, n, 0)),
        ],
        out_shape=[
            jax.ShapeDtypeStruct((NP, DIM), jnp.float32),
            jax.ShapeDtypeStruct((2, NP, 32), jnp.float32),
        ],
        name="node_update",
        interpret=_INTERPRET,
    )(agg, node, W2, b2, W3, b3, W1n)


BH = 400
REAL_H_BLOCKS = N // BH


def _head_body(node_ref, d1W_ref, d1b_ref, d2W_ref, d2b_ref, clsW_ref, clsb_ref,
               atomW_ref, atomb_ref, propW_ref, propb_ref,
               atoms_ref, cls_ref, prop_ref, acc_ref):
    n = pl.program_id(0)
    x = node_ref[...]
    a1 = jnp.dot(x, d1W_ref[...], preferred_element_type=jnp.float32) + d1b_ref[...]
    a1 = jnp.logaddexp(0.0, a1) - jnp.log(2.0)
    res = jnp.dot(a1, d2W_ref[...], preferred_element_type=jnp.float32) + d2b_ref[...]
    atoms_ref[...] = jnp.dot(res, atomW_ref[...], preferred_element_type=jnp.float32) + atomb_ref[...]

    @pl.when(n == 0)
    def _():
        acc_ref[...] = jnp.zeros_like(acc_ref)

    @pl.when(n < REAL_H_BLOCKS)
    def _():
        acc_ref[...] += jnp.sum(res, axis=0, keepdims=True)

    @pl.when(n == (NP // BH) - 1)
    def _():
        m = acc_ref[...] * (1.0 / N)
        cls_ref[...] = jnp.dot(m, clsW_ref[...], preferred_element_type=jnp.float32) + clsb_ref[...]
        prop_ref[...] = jnp.dot(m, propW_ref[...], preferred_element_type=jnp.float32) + propb_ref[...]


def _head(node, d1_W, d1_b, d2_W, d2_b, cls_W, cls_b, atom_W, atom_b, prop_W, prop_b):
    full = lambda a: pl.BlockSpec(a.shape, lambda n: (0,) * a.ndim)
    return pl.pallas_call(
        _head_body,
        grid=(NP // BH,),
        in_specs=[
            pl.BlockSpec((BH, DIM), lambda n: (n, 0)),
            full(d1_W), full(d1_b), full(d2_W), full(d2_b),
            full(cls_W), full(cls_b), full(atom_W), full(atom_b),
            full(prop_W), full(prop_b),
        ],
        out_specs=[
            pl.BlockSpec((BH, TYPE_NUM), lambda n: (n, 0)),
            pl.BlockSpec((1, 2000), lambda n: (0, 0)),
            pl.BlockSpec((1, 30), lambda n: (0, 0)),
        ],
        out_shape=[
            jax.ShapeDtypeStruct((NP, TYPE_NUM), jnp.float32),
            jax.ShapeDtypeStruct((1, 2000), jnp.float32),
            jax.ShapeDtypeStruct((1, 30), jnp.float32),
        ],
        scratch_shapes=[pltpu.VMEM((1, 256), jnp.float32)],
        name="mlp_head",
        interpret=_INTERPRET,
    )(node, d1_W, d1_b, d2_W, d2_b, cls_W, cls_b, atom_W, atom_b, prop_W, prop_b)


CH = 128
N_CH = 2
SUP = CH * N_CH
N_TILES = 16
E_PER_TILE = EP // N_TILES
N_SUP = E_PER_TILE // SUP
ROWS_PER_TILE = NP // N_TILES


def _make_edge_kernel(layer):
    mesh = plsc.VectorSubcoreMesh(core_axis_name="c", subcore_axis_name="s",
                                  num_cores=2, num_subcores=N_TILES)

    @functools.partial(
        pl.kernel,
        out_type=jax.ShapeDtypeStruct((2, NP, 32), jnp.float32),
        mesh=mesh,
        scratch_types=[
            pltpu.VMEM((N_CH, CH), jnp.int32),
            pltpu.VMEM((N_CH, CH), jnp.int32),
            pltpu.VMEM((SUP // 128, 32, 129), jnp.float32),
            pltpu.VMEM((SUP, 32), jnp.float32),
            pltpu.VMEM_SHARED((NP, 32), jnp.float32),
            pltpu.SemaphoreType.DMA,
        ],
        compiler_params=pltpu.CompilerParams(use_tc_tiling_on_sc=False,
                                             needs_layout_passes=False),
        name=f"sc_edge{layer}",
    )
    def ek(src_hbm, dst_hbm, h_hbm, tab_hbm, out_hbm,
           src_v, dst_v, h_v, g_v, agg_sh, sem):
        c = lax.axis_index("c")
        s = lax.axis_index("s")

        def zb(i, carry):
            g_v[i, pl.ds(0, 16)] = jnp.zeros((16,), jnp.float32)
            g_v[i, pl.ds(16, 16)] = jnp.zeros((16,), jnp.float32)
            return carry
        lax.fori_loop(0, CH, zb, 0)
        r_base = pl.multiple_of(s * ROWS_PER_TILE, CH)
        for t in range(ROWS_PER_TILE // CH):
            pltpu.sync_copy(g_v.at[pl.ds(0, CH)],
                            agg_sh.at[pl.ds(r_base + t * CH, CH)])
        plsc.subcore_barrier()

        e_base = s * E_PER_TILE

        def sup_body(g, carry):
            e0 = pl.multiple_of(e_base + g * SUP, SUP)
            row0 = pl.multiple_of(e0 // CH, N_CH)
            pltpu.sync_copy(src_hbm.at[pl.ds(row0, N_CH)], src_v)
            pltpu.sync_copy(dst_hbm.at[pl.ds(row0, N_CH)], dst_v)
            f0 = pl.multiple_of(c * 32, 32)
            s_idx = e0 // EB
            b0 = pl.multiple_of((e0 // 128) % (EB // 128), SUP // 128)
            pltpu.sync_copy(
                h_hbm.at[layer, s_idx, pl.ds(b0, SUP // 128), pl.ds(f0, 32)],
                h_v.at[:, :, pl.ds(0, 128)])
            descs = [
                pltpu.async_copy(tab_hbm.at[c].at[src_v.at[j]],
                                 g_v.at[pl.ds(j * CH, CH)], sem)
                for j in range(N_CH)
            ]
            for d in descs:
                d.wait()

            iota16 = lax.iota(jnp.int32, 16)
            iota16b = iota16 + 16
            for chunk in range(SUP // 128):
                hc = h_v.at[chunk]

                def mrow(j, carry2, _chunk=chunk):
                    ev = jnp.zeros((16,), jnp.int32) + j
                    h0 = plsc.load_gather(hc, [iota16, ev])
                    h1 = plsc.load_gather(hc, [iota16b, ev])
                    rr = _chunk * 128 + j
                    g_v[rr, pl.ds(0, 16)] = g_v[rr, pl.ds(0, 16)] * h0
                    g_v[rr, pl.ds(16, 16)] = g_v[rr, pl.ds(16, 16)] * h1
                    return carry2
                lax.fori_loop(0, 128, mrow, 0)

            for j in range(N_CH):
                pltpu.sync_copy(g_v.at[pl.ds(j * CH, CH)],
                                agg_sh.at[dst_v.at[j]], add=True)
            return carry
        lax.fori_loop(0, N_SUP, sup_body, 0)

        plsc.subcore_barrier()
        o_base = pl.multiple_of(s * ROWS_PER_TILE, ROWS_PER_TILE)
        pltpu.sync_copy(agg_sh.at[pl.ds(o_base, ROWS_PER_TILE)],
                        out_hbm.at[c, pl.ds(o_base, ROWS_PER_TILE)])

    return ek


_edge_kernel_cache = {}


def _edge_call(layer, src2d, dst2d, h_all, nn):
    if layer not in _edge_kernel_cache:
        _edge_kernel_cache[layer] = _make_edge_kernel(layer)
    return _edge_kernel_cache[layer](src2d, dst2d, h_all, nn)


def kernel(node_type, edge_index, distance, emb, conv_W1, cf_W1, cf_b1, cf_W2,
           cf_b2, conv_W2, conv_b2, conv_W3, conv_b3, d1_W, d1_b, d2_W, d2_b,
           cls_W, cls_b, atom_W, atom_b, prop_W, prop_b):
    i32 = jnp.int32
    nt_pad = jnp.concatenate(
        [node_type.astype(i32), jnp.zeros((NP - N,), i32)]).reshape(NP, 1)
    src2d = jnp.concatenate(
        [edge_index[0].astype(i32), jnp.zeros((EP - E,), i32)]).reshape(EP // CH, CH)
    dst2d = jnp.concatenate(
        [edge_index[1].astype(i32), jnp.zeros((EP - E,), i32)]).reshape(EP // CH, CH)
    d_pad = jnp.concatenate(
        [distance, jnp.zeros((EP - E,), jnp.float32)]).reshape(1, EP)

    h_all = _compute_h(d_pad, jnp.swapaxes(cf_W1, 1, 2),
                       cf_b1.reshape(N_CONV, DIM, 1),
                       jnp.swapaxes(cf_W2, 1, 2),
                       cf_b2.reshape(N_CONV, DIM, 1))
    node, nn = _embed(nt_pad, emb, conv_W1[0])

    for i in range(N_CONV):
        agg = _edge_call(i, src2d, dst2d, h_all, nn)
        W1n = conv_W1[i + 1] if i + 1 < N_CONV else conv_W1[0]
        node, nn = _update(agg, node, conv_W2[i], conv_b2[i].reshape(1, DIM),
                           conv_W3[i], conv_b3[i].reshape(1, DIM), W1n)

    atoms, cls_p, prop_p = _head(node, d1_W, d1_b.reshape(1, 256), d2_W,
                                 d2_b.reshape(1, 256), cls_W, cls_b.reshape(1, 2000),
                                 atom_W, atom_b.reshape(1, TYPE_NUM), prop_W,
                                 prop_b.reshape(1, 30))
    return (atoms[:N], cls_p, prop_p)


# speedup vs baseline: 2.8065x; 1.3866x over previous
import functools

import numpy as np

import jax
import jax.numpy as jnp
from jax import lax
from jax.experimental import pallas as pl
from jax.experimental.pallas import tpu as pltpu
from jax.experimental.pallas import tpu_sc as plsc

N = 50000
E = 800000
DIM = 64
TYPE_NUM = 100
N_CONV = 3
CUTOFF = 5.0
N_CENTERS = 5
GAP = CUTOFF / (N_CENTERS - 1)
_CENTERS_NP = np.linspace(0.0, CUTOFF, N_CENTERS).astype(np.float32)

NP = 51200
EP = 819200

_INTERPRET = False

EB = 8192


def _softplus_b05(x):
    return 2.0 * jnp.logaddexp(0.0, 0.5 * x)


def _h_body(d_ref, w1t_ref, b1_ref, w2t_ref, b2_ref, out_ref):
    e = pl.program_id(1)
    d = d_ref[...]
    rbf_t = jnp.concatenate(
        [jnp.exp((-1.0 / GAP) * (d - float(c)) ** 2) for c in _CENTERS_NP],
        axis=0)
    t = jnp.dot(w1t_ref[0], rbf_t, preferred_element_type=jnp.float32) + b1_ref[0]
    t = _softplus_b05(t)
    h = jnp.dot(w2t_ref[0], t, preferred_element_type=jnp.float32) + b2_ref[0]
    ids = e * EB + lax.broadcasted_iota(jnp.int32, (1, EB), 1)
    hm = jnp.where(ids < E, h, 0.0)
    for b in range(EB // 128):
        out_ref[0, 0, b] = hm[:, 128 * b:128 * (b + 1)]


def _compute_h(d_pad, cf_W1T, cf_b1, cf_W2T, cf_b2):
    return pl.pallas_call(
        _h_body,
        grid=(N_CONV, EP // EB),
        in_specs=[
            pl.BlockSpec((1, EB), lambda i, e: (0, e)),
            pl.BlockSpec((1, DIM, N_CENTERS), lambda i, e: (i, 0, 0)),
            pl.BlockSpec((1, DIM, 1), lambda i, e: (i, 0, 0)),
            pl.BlockSpec((1, DIM, DIM), lambda i, e: (i, 0, 0)),
            pl.BlockSpec((1, DIM, 1), lambda i, e: (i, 0, 0)),
        ],
        out_specs=pl.BlockSpec((1, 1, EB // 128, DIM, 128),
                               lambda i, e: (i, e, 0, 0, 0)),
        out_shape=jax.ShapeDtypeStruct(
            (N_CONV, EP // EB, EB // 128, DIM, 128), jnp.float32),
        name="h_filter",
        interpret=_INTERPRET,
    )(d_pad, cf_W1T, cf_b1, cf_W2T, cf_b2)


BN = 512


def _embed_body(nt_ref, emb_ref, w_ref, node_ref, nn_ref):
    nt = nt_ref[...]
    oh = (nt == lax.broadcasted_iota(jnp.int32, (BN, TYPE_NUM), 1)).astype(jnp.float32)
    nodev = jnp.dot(oh, emb_ref[...], preferred_element_type=jnp.float32)
    node_ref[...] = nodev
    nn = jnp.dot(nodev, w_ref[...], preferred_element_type=jnp.float32)
    nn_ref[0] = nn[:, :32]
    nn_ref[1] = nn[:, 32:]


def _embed(nt_pad, emb, W1_0):
    return pl.pallas_call(
        _embed_body,
        grid=(NP // BN,),
        in_specs=[
            pl.BlockSpec((BN, 1), lambda n: (n, 0)),
            pl.BlockSpec((TYPE_NUM, DIM), lambda n: (0, 0)),
            pl.BlockSpec((DIM, DIM), lambda n: (0, 0)),
        ],
        out_specs=[
            pl.BlockSpec((BN, DIM), lambda n: (n, 0)),
            pl.BlockSpec((2, BN, 32), lambda n: (0, n, 0)),
        ],
        out_shape=[
            jax.ShapeDtypeStruct((NP, DIM), jnp.float32),
            jax.ShapeDtypeStruct((2, NP, 32), jnp.float32),
        ],
        name="embed_w1",
        interpret=_INTERPRET,
    )(nt_pad, emb, W1_0)


def _update_body(agg_ref, node_ref, w2_ref, b2_ref, w3_ref, b3_ref, w1n_ref,
                 node_out_ref, nn_ref):
    aggc = jnp.concatenate([agg_ref[0], agg_ref[1]], axis=1)
    cf1 = jnp.dot(aggc, w2_ref[...], preferred_element_type=jnp.float32) + b2_ref[...]
    a = _softplus_b05(cf1)
    upd = jnp.dot(a, w3_ref[...], preferred_element_type=jnp.float32) + b3_ref[...]
    nodev = node_ref[...] + upd
    node_out_ref[...] = nodev
    nn = jnp.dot(nodev, w1n_ref[...], preferred_element_type=jnp.float32)
    nn_ref[0] = nn[:, :32]
    nn_ref[1] = nn[:, 32:]


def _update(agg, node, W2, b2, W3, b3, W1n):
    return pl.pallas_call(
        _update_body,
        grid=(NP // BN,),
        in_specs=[
            pl.BlockSpec((2, BN, 32), lambda n: (0, n, 0)),
            pl.BlockSpec((BN, DIM), lambda n: (n, 0)),
            pl.BlockSpec((DIM, DIM), lambda n: (0, 0)),
            pl.BlockSpec((1, DIM), lambda n: (0, 0)),
            pl.BlockSpec((DIM, DIM), lambda n: (0, 0)),
            pl.BlockSpec((1, DIM), lambda n: (0, 0)),
            pl.BlockSpec((DIM, DIM), lambda n: (0, 0)),
        ],
        out_specs=[
            pl.BlockSpec((BN, DIM), lambda n: (n, 0)),
            pl.BlockSpec((2, BN, 32), lambda n: (0, n, 0)),
        ],
        out_shape=[
            jax.ShapeDtypeStruct((NP, DIM), jnp.float32),
            jax.ShapeDtypeStruct((2, NP, 32), jnp.float32),
        ],
        name="node_update",
        interpret=_INTERPRET,
    )(agg, node, W2, b2, W3, b3, W1n)


BH = 400
REAL_H_BLOCKS = N // BH


def _head_body(node_ref, d1W_ref, d1b_ref, d2W_ref, d2b_ref, clsW_ref, clsb_ref,
               atomW_ref, atomb_ref, propW_ref, propb_ref,
               atoms_ref, cls_ref, prop_ref, acc_ref):
    n = pl.program_id(0)
    x = node_ref[...]
    a1 = jnp.dot(x, d1W_ref[...], preferred_element_type=jnp.float32) + d1b_ref[...]
    a1 = jnp.logaddexp(0.0, a1) - jnp.log(2.0)
    res = jnp.dot(a1, d2W_ref[...], preferred_element_type=jnp.float32) + d2b_ref[...]
    atoms_ref[...] = jnp.dot(res, atomW_ref[...], preferred_element_type=jnp.float32) + atomb_ref[...]

    @pl.when(n == 0)
    def _():
        acc_ref[...] = jnp.zeros_like(acc_ref)

    @pl.when(n < REAL_H_BLOCKS)
    def _():
        acc_ref[...] += jnp.sum(res, axis=0, keepdims=True)

    @pl.when(n == (NP // BH) - 1)
    def _():
        m = acc_ref[...] * (1.0 / N)
        cls_ref[...] = jnp.dot(m, clsW_ref[...], preferred_element_type=jnp.float32) + clsb_ref[...]
        prop_ref[...] = jnp.dot(m, propW_ref[...], preferred_element_type=jnp.float32) + propb_ref[...]


def _head(node, d1_W, d1_b, d2_W, d2_b, cls_W, cls_b, atom_W, atom_b, prop_W, prop_b):
    full = lambda a: pl.BlockSpec(a.shape, lambda n: (0,) * a.ndim)
    return pl.pallas_call(
        _head_body,
        grid=(NP // BH,),
        in_specs=[
            pl.BlockSpec((BH, DIM), lambda n: (n, 0)),
            full(d1_W), full(d1_b), full(d2_W), full(d2_b),
            full(cls_W), full(cls_b), full(atom_W), full(atom_b),
            full(prop_W), full(prop_b),
        ],
        out_specs=[
            pl.BlockSpec((BH, TYPE_NUM), lambda n: (n, 0)),
            pl.BlockSpec((1, 2000), lambda n: (0, 0)),
            pl.BlockSpec((1, 30), lambda n: (0, 0)),
        ],
        out_shape=[
            jax.ShapeDtypeStruct((NP, TYPE_NUM), jnp.float32),
            jax.ShapeDtypeStruct((1, 2000), jnp.float32),
            jax.ShapeDtypeStruct((1, 30), jnp.float32),
        ],
        scratch_shapes=[pltpu.VMEM((1, 256), jnp.float32)],
        name="mlp_head",
        interpret=_INTERPRET,
    )(node, d1_W, d1_b, d2_W, d2_b, cls_W, cls_b, atom_W, atom_b, prop_W, prop_b)


SUP = 128
N_TILES = 16
E_PER_TILE = EP // N_TILES
N_SUP = E_PER_TILE // SUP
ROWS_PER_TILE = NP // N_TILES


def _make_edge_kernel(layer):
    mesh = plsc.VectorSubcoreMesh(core_axis_name="c", subcore_axis_name="s",
                                  num_cores=2, num_subcores=N_TILES)

    @functools.partial(
        pl.kernel,
        out_type=jax.ShapeDtypeStruct((2, NP, 32), jnp.float32),
        mesh=mesh,
        scratch_types=[
            pltpu.VMEM((1, SUP), jnp.int32),
            pltpu.VMEM((1, SUP), jnp.int32),
            pltpu.VMEM((1, SUP), jnp.int32),
            pltpu.VMEM((1, SUP), jnp.int32),
            pltpu.VMEM((1, SUP), jnp.int32),
            pltpu.VMEM((1, SUP), jnp.int32),
            pltpu.VMEM((32, 129), jnp.float32),
            pltpu.VMEM((32, 129), jnp.float32),
            pltpu.VMEM((SUP, 32), jnp.float32),
            pltpu.VMEM((SUP, 32), jnp.float32),
            pltpu.VMEM_SHARED((NP, 32), jnp.float32),
            pltpu.SemaphoreType.DMA, pltpu.SemaphoreType.DMA,
            pltpu.SemaphoreType.DMA, pltpu.SemaphoreType.DMA,
            pltpu.SemaphoreType.DMA, pltpu.SemaphoreType.DMA,
        ],
        compiler_params=pltpu.CompilerParams(use_tc_tiling_on_sc=False,
                                             needs_layout_passes=False),
        name=f"sc_edge{layer}",
    )
    def ek(src_hbm, dst_hbm, h_hbm, tab_hbm, out_hbm,
           srcA, srcB, dstA, dstB, dstC, dstD, hA, hB, gA, gB,
           agg_sh, linA, linB, gatA, gatB, scA, scB):
        c = lax.axis_index("c")
        s = lax.axis_index("s")
        srcs = [srcA, srcB]
        dsts = [dstA, dstB, dstC, dstD]
        hs = [hA, hB]
        gs = [gA, gB]
        lins = [linA, linB]
        gats = [gatA, gatB]
        scs = [scA, scB]
        f0 = pl.multiple_of(c * 32, 32)
        e_base = s * E_PER_TILE
        row_base = e_base // SUP

        def zb(i, carry):
            gA[i, pl.ds(0, 16)] = jnp.zeros((16,), jnp.float32)
            gA[i, pl.ds(16, 16)] = jnp.zeros((16,), jnp.float32)
            return carry
        lax.fori_loop(0, SUP, zb, 0)
        r_base = pl.multiple_of(s * ROWS_PER_TILE, SUP)
        for t in range(ROWS_PER_TILE // SUP):
            pltpu.sync_copy(gA.at[pl.ds(0, SUP)],
                            agg_sh.at[pl.ds(r_base + t * SUP, SUP)])
        plsc.subcore_barrier()

        def lin_start(gi, sp, dj):
            row0 = row_base + jnp.minimum(gi, N_SUP - 1)
            pltpu.async_copy(src_hbm.at[pl.ds(row0, 1)], srcs[sp], lins[sp])
            pltpu.async_copy(dst_hbm.at[pl.ds(row0, 1)], dsts[dj], lins[sp])
            s_idx = row0 // (EB // SUP)
            b0 = row0 % (EB // SUP)
            pltpu.async_copy(h_hbm.at[layer, s_idx, b0, pl.ds(f0, 32)],
                             hs[sp].at[:, pl.ds(0, 128)], lins[sp])

        def lin_wait(sp, dj):
            pltpu.make_async_copy(src_hbm.at[pl.ds(0, 1)], srcs[sp], lins[sp]).wait()
            pltpu.make_async_copy(dst_hbm.at[pl.ds(0, 1)], dsts[dj], lins[sp]).wait()
            pltpu.make_async_copy(h_hbm.at[layer, 0, 0, pl.ds(0, 32)],
                                  hs[sp].at[:, pl.ds(0, 128)], lins[sp]).wait()

        def gat_start(sp):
            pltpu.async_copy(tab_hbm.at[c].at[srcs[sp].at[0]], gs[sp], gats[sp])

        def gat_wait(sp):
            pltpu.make_async_copy(tab_hbm.at[c, pl.ds(0, SUP)], gs[sp],
                                  gats[sp]).wait()

        def sc_start(dj, sp):
            pltpu.async_copy(gs[sp], agg_sh.at[dsts[dj].at[0]], scs[sp], add=True)

        def sc_wait(sp):
            pltpu.make_async_copy(gs[sp], agg_sh.at[pl.ds(0, SUP)], scs[sp]).wait()

        iota16 = lax.iota(jnp.int32, 16)
        iota16b = iota16 + 16

        def multiply(sp):
            hc = hs[sp]
            gv = gs[sp]

            def mrow(j, carry2):
                ev = jnp.zeros((16,), jnp.int32) + j
                h0 = plsc.load_gather(hc, [iota16, ev])
                h1 = plsc.load_gather(hc, [iota16b, ev])
                gv[j, pl.ds(0, 16)] = gv[j, pl.ds(0, 16)] * h0
                gv[j, pl.ds(16, 16)] = gv[j, pl.ds(16, 16)] * h1
                return carry2
            lax.fori_loop(0, SUP, mrow, 0)

        lin_start(0, 0, 0)
        lin_start(1, 1, 1)
        lin_wait(0, 0)
        gat_start(0)

        def round_body(t, carry):
            g0 = t * 4
            for j in range(4):
                p = j % 2
                q = 1 - p
                gi = g0 + j
                lin_wait(q, (j + 1) % 4)
                if j == 0:
                    @pl.when(t > 0)
                    def _():
                        sc_wait(q)
                else:
                    sc_wait(q)
                gat_start(q)
                gat_wait(p)
                multiply(p)
                sc_start(j, p)
                lin_start(gi + 2, p, (j + 2) % 4)
            return carry
        lax.fori_loop(0, N_SUP // 4, round_body, 0)

        sc_wait(1)
        gat_wait(0)
        lin_wait(1, 1)

        plsc.subcore_barrier()
        o_base = pl.multiple_of(s * ROWS_PER_TILE, ROWS_PER_TILE)
        pltpu.sync_copy(agg_sh.at[pl.ds(o_base, ROWS_PER_TILE)],
                        out_hbm.at[c, pl.ds(o_base, ROWS_PER_TILE)])

    return ek


_edge_kernel_cache = {}


def _edge_call(layer, src2d, dst2d, h_all, nn):
    if layer not in _edge_kernel_cache:
        _edge_kernel_cache[layer] = _make_edge_kernel(layer)
    return _edge_kernel_cache[layer](src2d, dst2d, h_all, nn)


def kernel(node_type, edge_index, distance, emb, conv_W1, cf_W1, cf_b1, cf_W2,
           cf_b2, conv_W2, conv_b2, conv_W3, conv_b3, d1_W, d1_b, d2_W, d2_b,
           cls_W, cls_b, atom_W, atom_b, prop_W, prop_b):
    i32 = jnp.int32
    nt_pad = jnp.concatenate(
        [node_type.astype(i32), jnp.zeros((NP - N,), i32)]).reshape(NP, 1)
    src2d = jnp.concatenate(
        [edge_index[0].astype(i32), jnp.zeros((EP - E,), i32)]).reshape(EP // SUP, SUP)
    dst2d = jnp.concatenate(
        [edge_index[1].astype(i32), jnp.zeros((EP - E,), i32)]).reshape(EP // SUP, SUP)
    d_pad = jnp.concatenate(
        [distance, jnp.zeros((EP - E,), jnp.float32)]).reshape(1, EP)

    h_all = _compute_h(d_pad, jnp.swapaxes(cf_W1, 1, 2),
                       cf_b1.reshape(N_CONV, DIM, 1),
                       jnp.swapaxes(cf_W2, 1, 2),
                       cf_b2.reshape(N_CONV, DIM, 1))
    node, nn = _embed(nt_pad, emb, conv_W1[0])

    for i in range(N_CONV):
        agg = _edge_call(i, src2d, dst2d, h_all, nn)
        W1n = conv_W1[i + 1] if i + 1 < N_CONV else conv_W1[0]
        node, nn = _update(agg, node, conv_W2[i], conv_b2[i].reshape(1, DIM),
                           conv_W3[i], conv_b3[i].reshape(1, DIM), W1n)

    atoms, cls_p, prop_p = _head(node, d1_W, d1_b.reshape(1, 256), d2_W,
                                 d2_b.reshape(1, 256), cls_W, cls_b.reshape(1, 2000),
                                 atom_W, atom_b.reshape(1, TYPE_NUM), prop_W,
                                 prop_b.reshape(1, 30))
    return (atoms[:N], cls_p, prop_p)


# speedup vs baseline: 3.1435x; 1.1201x over previous
import functools

import numpy as np

import jax
import jax.numpy as jnp
from jax import lax
from jax.experimental import pallas as pl
from jax.experimental.pallas import tpu as pltpu
from jax.experimental.pallas import tpu_sc as plsc

N = 50000
E = 800000
DIM = 64
TYPE_NUM = 100
N_CONV = 3
CUTOFF = 5.0
N_CENTERS = 5
GAP = CUTOFF / (N_CENTERS - 1)
_CENTERS_NP = np.linspace(0.0, CUTOFF, N_CENTERS).astype(np.float32)

NP = 51200
EP = 819200

_INTERPRET = False

EB = 8192


def _softplus_b05(x):
    return 2.0 * jnp.logaddexp(0.0, 0.5 * x)


def _h_body(d_ref, w1t_ref, b1_ref, w2t_ref, b2_ref, out_ref):
    e = pl.program_id(0)
    d = d_ref[...]
    rbf_t = jnp.concatenate(
        [jnp.exp((-1.0 / GAP) * (d - float(c)) ** 2) for c in _CENTERS_NP],
        axis=0)
    t = jnp.dot(w1t_ref[...], rbf_t, preferred_element_type=jnp.float32) + b1_ref[...]
    t = _softplus_b05(t)
    h = jnp.dot(w2t_ref[...], t, preferred_element_type=jnp.float32) + b2_ref[...]
    ids = e * EB + lax.broadcasted_iota(jnp.int32, (1, EB), 1)
    hm = jnp.where(ids < E, h, 0.0)
    for b in range(EB // 128):
        out_ref[0, b] = hm[:, 128 * b:128 * (b + 1)]


def _compute_h(d_pad, cf_W1T_i, cf_b1_i, cf_W2T_i, cf_b2_i):
    return pl.pallas_call(
        _h_body,
        grid=(EP // EB,),
        in_specs=[
            pl.BlockSpec((1, EB), lambda e: (0, e)),
            pl.BlockSpec((DIM, N_CENTERS), lambda e: (0, 0)),
            pl.BlockSpec((DIM, 1), lambda e: (0, 0)),
            pl.BlockSpec((DIM, DIM), lambda e: (0, 0)),
            pl.BlockSpec((DIM, 1), lambda e: (0, 0)),
        ],
        out_specs=pl.BlockSpec((1, EB // 128, DIM, 128),
                               lambda e: (e, 0, 0, 0)),
        out_shape=jax.ShapeDtypeStruct(
            (EP // EB, EB // 128, DIM, 128), jnp.float32),
        name="h_filter",
        interpret=_INTERPRET,
    )(d_pad, cf_W1T_i, cf_b1_i, cf_W2T_i, cf_b2_i)


BN = 512


def _embed_body(nt_ref, emb_ref, w_ref, node_ref, nn_ref):
    nt = nt_ref[...]
    oh = (nt == lax.broadcasted_iota(jnp.int32, (BN, TYPE_NUM), 1)).astype(jnp.float32)
    nodev = jnp.dot(oh, emb_ref[...], preferred_element_type=jnp.float32)
    node_ref[...] = nodev
    nn = jnp.dot(nodev, w_ref[...], preferred_element_type=jnp.float32)
    nn_ref[0] = nn[:, :32]
    nn_ref[1] = nn[:, 32:]


def _embed(nt_pad, emb, W1_0):
    return pl.pallas_call(
        _embed_body,
        grid=(NP // BN,),
        in_specs=[
            pl.BlockSpec((BN, 1), lambda n: (n, 0)),
            pl.BlockSpec((TYPE_NUM, DIM), lambda n: (0, 0)),
            pl.BlockSpec((DIM, DIM), lambda n: (0, 0)),
        ],
        out_specs=[
            pl.BlockSpec((BN, DIM), lambda n: (n, 0)),
            pl.BlockSpec((2, BN, 32), lambda n: (0, n, 0)),
        ],
        out_shape=[
            jax.ShapeDtypeStruct((NP, DIM), jnp.float32),
            jax.ShapeDtypeStruct((2, NP, 32), jnp.float32),
        ],
        name="embed_w1",
        interpret=_INTERPRET,
    )(nt_pad, emb, W1_0)


def _update_body(agg_ref, node_ref, w2_ref, b2_ref, w3_ref, b3_ref, w1n_ref,
                 node_out_ref, nn_ref):
    aggc = jnp.concatenate([agg_ref[0], agg_ref[1]], axis=1)
    cf1 = jnp.dot(aggc, w2_ref[...], preferred_element_type=jnp.float32) + b2_ref[...]
    a = _softplus_b05(cf1)
    upd = jnp.dot(a, w3_ref[...], preferred_element_type=jnp.float32) + b3_ref[...]
    nodev = node_ref[...] + upd
    node_out_ref[...] = nodev
    nn = jnp.dot(nodev, w1n_ref[...], preferred_element_type=jnp.float32)
    nn_ref[0] = nn[:, :32]
    nn_ref[1] = nn[:, 32:]


def _update(agg, node, W2, b2, W3, b3, W1n):
    return pl.pallas_call(
        _update_body,
        grid=(NP // BN,),
        in_specs=[
            pl.BlockSpec((2, BN, 32), lambda n: (0, n, 0)),
            pl.BlockSpec((BN, DIM), lambda n: (n, 0)),
            pl.BlockSpec((DIM, DIM), lambda n: (0, 0)),
            pl.BlockSpec((1, DIM), lambda n: (0, 0)),
            pl.BlockSpec((DIM, DIM), lambda n: (0, 0)),
            pl.BlockSpec((1, DIM), lambda n: (0, 0)),
            pl.BlockSpec((DIM, DIM), lambda n: (0, 0)),
        ],
        out_specs=[
            pl.BlockSpec((BN, DIM), lambda n: (n, 0)),
            pl.BlockSpec((2, BN, 32), lambda n: (0, n, 0)),
        ],
        out_shape=[
            jax.ShapeDtypeStruct((NP, DIM), jnp.float32),
            jax.ShapeDtypeStruct((2, NP, 32), jnp.float32),
        ],
        name="node_update",
        interpret=_INTERPRET,
    )(agg, node, W2, b2, W3, b3, W1n)


BH = 400
REAL_H_BLOCKS = N // BH


def _head_body(node_ref, d1W_ref, d1b_ref, d2W_ref, d2b_ref, clsW_ref, clsb_ref,
               atomW_ref, atomb_ref, propW_ref, propb_ref,
               atoms_ref, cls_ref, prop_ref, acc_ref):
    n = pl.program_id(0)
    x = node_ref[...]
    a1 = jnp.dot(x, d1W_ref[...], preferred_element_type=jnp.float32) + d1b_ref[...]
    a1 = jnp.logaddexp(0.0, a1) - jnp.log(2.0)
    res = jnp.dot(a1, d2W_ref[...], preferred_element_type=jnp.float32) + d2b_ref[...]
    atoms_ref[...] = jnp.dot(res, atomW_ref[...], preferred_element_type=jnp.float32) + atomb_ref[...]

    @pl.when(n == 0)
    def _():
        acc_ref[...] = jnp.zeros_like(acc_ref)

    @pl.when(n < REAL_H_BLOCKS)
    def _():
        acc_ref[...] += jnp.sum(res, axis=0, keepdims=True)

    @pl.when(n == (NP // BH) - 1)
    def _():
        m = acc_ref[...] * (1.0 / N)
        cls_ref[...] = jnp.dot(m, clsW_ref[...], preferred_element_type=jnp.float32) + clsb_ref[...]
        prop_ref[...] = jnp.dot(m, propW_ref[...], preferred_element_type=jnp.float32) + propb_ref[...]


def _head(node, d1_W, d1_b, d2_W, d2_b, cls_W, cls_b, atom_W, atom_b, prop_W, prop_b):
    full = lambda a: pl.BlockSpec(a.shape, lambda n: (0,) * a.ndim)
    return pl.pallas_call(
        _head_body,
        grid=(NP // BH,),
        in_specs=[
            pl.BlockSpec((BH, DIM), lambda n: (n, 0)),
            full(d1_W), full(d1_b), full(d2_W), full(d2_b),
            full(cls_W), full(cls_b), full(atom_W), full(atom_b),
            full(prop_W), full(prop_b),
        ],
        out_specs=[
            pl.BlockSpec((BH, TYPE_NUM), lambda n: (n, 0)),
            pl.BlockSpec((1, 2000), lambda n: (0, 0)),
            pl.BlockSpec((1, 30), lambda n: (0, 0)),
        ],
        out_shape=[
            jax.ShapeDtypeStruct((NP, TYPE_NUM), jnp.float32),
            jax.ShapeDtypeStruct((1, 2000), jnp.float32),
            jax.ShapeDtypeStruct((1, 30), jnp.float32),
        ],
        scratch_shapes=[pltpu.VMEM((1, 256), jnp.float32)],
        name="mlp_head",
        interpret=_INTERPRET,
    )(node, d1_W, d1_b, d2_W, d2_b, cls_W, cls_b, atom_W, atom_b, prop_W, prop_b)


SUP = 128
N_TILES = 16
E_PER_TILE = EP // N_TILES
N_SUP = E_PER_TILE // SUP
ROWS_PER_TILE = NP // N_TILES


def _make_edge_kernel(layer):
    mesh = plsc.VectorSubcoreMesh(core_axis_name="c", subcore_axis_name="s",
                                  num_cores=2, num_subcores=N_TILES)

    @functools.partial(
        pl.kernel,
        out_type=jax.ShapeDtypeStruct((2, NP, 32), jnp.float32),
        mesh=mesh,
        scratch_types=[
            pltpu.VMEM((1, SUP), jnp.int32),
            pltpu.VMEM((1, SUP), jnp.int32),
            pltpu.VMEM((1, SUP), jnp.int32),
            pltpu.VMEM((1, SUP), jnp.int32),
            pltpu.VMEM((1, SUP), jnp.int32),
            pltpu.VMEM((1, SUP), jnp.int32),
            pltpu.VMEM((32, 129), jnp.float32),
            pltpu.VMEM((32, 129), jnp.float32),
            pltpu.VMEM((SUP, 32), jnp.float32),
            pltpu.VMEM((SUP, 32), jnp.float32),
            pltpu.VMEM_SHARED((NP, 32), jnp.float32),
            pltpu.SemaphoreType.DMA, pltpu.SemaphoreType.DMA,
            pltpu.SemaphoreType.DMA, pltpu.SemaphoreType.DMA,
            pltpu.SemaphoreType.DMA, pltpu.SemaphoreType.DMA,
        ],
        compiler_params=pltpu.CompilerParams(use_tc_tiling_on_sc=False,
                                             needs_layout_passes=False),
        name=f"sc_edge{layer}",
    )
    def ek(src_hbm, dst_hbm, h_hbm, tab_hbm, out_hbm,
           srcA, srcB, dstA, dstB, dstC, dstD, hA, hB, gA, gB,
           agg_sh, linA, linB, gatA, gatB, scA, scB):
        c = lax.axis_index("c")
        s = lax.axis_index("s")
        srcs = [srcA, srcB]
        dsts = [dstA, dstB, dstC, dstD]
        hs = [hA, hB]
        gs = [gA, gB]
        lins = [linA, linB]
        gats = [gatA, gatB]
        scs = [scA, scB]
        f0 = pl.multiple_of(c * 32, 32)
        e_base = s * E_PER_TILE
        row_base = e_base // SUP

        def zb(i, carry):
            gA[i, pl.ds(0, 16)] = jnp.zeros((16,), jnp.float32)
            gA[i, pl.ds(16, 16)] = jnp.zeros((16,), jnp.float32)
            return carry
        lax.fori_loop(0, SUP, zb, 0)
        r_base = pl.multiple_of(s * ROWS_PER_TILE, SUP)
        for t in range(ROWS_PER_TILE // SUP):
            pltpu.sync_copy(gA.at[pl.ds(0, SUP)],
                            agg_sh.at[pl.ds(r_base + t * SUP, SUP)])
        plsc.subcore_barrier()

        def lin_start(gi, sp, dj):
            row0 = row_base + jnp.minimum(gi, N_SUP - 1)
            pltpu.async_copy(src_hbm.at[pl.ds(row0, 1)], srcs[sp], lins[sp])
            pltpu.async_copy(dst_hbm.at[pl.ds(row0, 1)], dsts[dj], lins[sp])
            s_idx = row0 // (EB // SUP)
            b0 = row0 % (EB // SUP)
            pltpu.async_copy(h_hbm.at[s_idx, b0, pl.ds(f0, 32)],
                             hs[sp].at[:, pl.ds(0, 128)], lins[sp])

        def lin_wait(sp, dj):
            pltpu.make_async_copy(src_hbm.at[pl.ds(0, 1)], srcs[sp], lins[sp]).wait()
            pltpu.make_async_copy(dst_hbm.at[pl.ds(0, 1)], dsts[dj], lins[sp]).wait()
            pltpu.make_async_copy(h_hbm.at[0, 0, pl.ds(0, 32)],
                                  hs[sp].at[:, pl.ds(0, 128)], lins[sp]).wait()

        def gat_start(sp):
            pltpu.async_copy(tab_hbm.at[c].at[srcs[sp].at[0]], gs[sp], gats[sp])

        def gat_wait(sp):
            pltpu.make_async_copy(tab_hbm.at[c, pl.ds(0, SUP)], gs[sp],
                                  gats[sp]).wait()

        def sc_start(dj, sp):
            pltpu.async_copy(gs[sp], agg_sh.at[dsts[dj].at[0]], scs[sp], add=True)

        def sc_wait(sp):
            pltpu.make_async_copy(gs[sp], agg_sh.at[pl.ds(0, SUP)], scs[sp]).wait()

        iota16 = lax.iota(jnp.int32, 16)
        iota16b = iota16 + 16

        def multiply(sp):
            hc = hs[sp]
            gv = gs[sp]

            def mrow(r, carry2):
                for u in range(4):
                    j = r * 4 + u
                    ev = jnp.zeros((16,), jnp.int32) + j
                    h0 = plsc.load_gather(hc, [iota16, ev])
                    h1 = plsc.load_gather(hc, [iota16b, ev])
                    gv[j, pl.ds(0, 16)] = gv[j, pl.ds(0, 16)] * h0
                    gv[j, pl.ds(16, 16)] = gv[j, pl.ds(16, 16)] * h1
                return carry2
            lax.fori_loop(0, SUP // 4, mrow, 0)

        lin_start(0, 0, 0)
        lin_start(1, 1, 1)
        lin_wait(0, 0)
        gat_start(0)

        def round_body(t, carry):
            g0 = t * 4
            for j in range(4):
                p = j % 2
                q = 1 - p
                gi = g0 + j
                lin_wait(q, (j + 1) % 4)
                if j == 0:
                    @pl.when(t > 0)
                    def _():
                        sc_wait(q)
                else:
                    sc_wait(q)
                gat_start(q)
                gat_wait(p)
                multiply(p)
                sc_start(j, p)
                lin_start(gi + 2, p, (j + 2) % 4)
            return carry
        lax.fori_loop(0, N_SUP // 4, round_body, 0)

        sc_wait(1)
        gat_wait(0)
        lin_wait(1, 1)

        plsc.subcore_barrier()
        o_base = pl.multiple_of(s * ROWS_PER_TILE, ROWS_PER_TILE)
        pltpu.sync_copy(agg_sh.at[pl.ds(o_base, ROWS_PER_TILE)],
                        out_hbm.at[c, pl.ds(o_base, ROWS_PER_TILE)])

    return ek


_edge_kernel_cache = {}


def _edge_call(layer, src2d, dst2d, h_all, nn):
    if layer not in _edge_kernel_cache:
        _edge_kernel_cache[layer] = _make_edge_kernel(layer)
    return _edge_kernel_cache[layer](src2d, dst2d, h_all, nn)


def kernel(node_type, edge_index, distance, emb, conv_W1, cf_W1, cf_b1, cf_W2,
           cf_b2, conv_W2, conv_b2, conv_W3, conv_b3, d1_W, d1_b, d2_W, d2_b,
           cls_W, cls_b, atom_W, atom_b, prop_W, prop_b):
    i32 = jnp.int32
    nt_pad = jnp.concatenate(
        [node_type.astype(i32), jnp.zeros((NP - N,), i32)]).reshape(NP, 1)
    src2d = jnp.concatenate(
        [edge_index[0].astype(i32), jnp.zeros((EP - E,), i32)]).reshape(EP // SUP, SUP)
    dst2d = jnp.concatenate(
        [edge_index[1].astype(i32), jnp.zeros((EP - E,), i32)]).reshape(EP // SUP, SUP)
    d_pad = jnp.concatenate(
        [distance, jnp.zeros((EP - E,), jnp.float32)]).reshape(1, EP)

    w1t = jnp.swapaxes(cf_W1, 1, 2)
    w2t = jnp.swapaxes(cf_W2, 1, 2)
    b1c = cf_b1.reshape(N_CONV, DIM, 1)
    b2c = cf_b2.reshape(N_CONV, DIM, 1)
    h_i = _compute_h(d_pad, w1t[0], b1c[0], w2t[0], b2c[0])
    node, nn = _embed(nt_pad, emb, conv_W1[0])

    for i in range(N_CONV):
        agg = _edge_call(i, src2d, dst2d, h_i, nn)
        if i + 1 < N_CONV:
            h_i = _compute_h(d_pad, w1t[i + 1], b1c[i + 1], w2t[i + 1], b2c[i + 1])
        W1n = conv_W1[i + 1] if i + 1 < N_CONV else conv_W1[0]
        node, nn = _update(agg, node, conv_W2[i], conv_b2[i].reshape(1, DIM),
                           conv_W3[i], conv_b3[i].reshape(1, DIM), W1n)

    atoms, cls_p, prop_p = _head(node, d1_W, d1_b.reshape(1, 256), d2_W,
                                 d2_b.reshape(1, 256), cls_W, cls_b.reshape(1, 2000),
                                 atom_W, atom_b.reshape(1, TYPE_NUM), prop_W,
                                 prop_b.reshape(1, 30))
    return (atoms[:N], cls_p, prop_p)
